# Initial kernel scaffold; baseline (speedup 1.0000x reference)
import jax
import jax.numpy as jnp
from jax.experimental import pallas as pl


def kernel(slices, inputs, W1, b1, W2, b2):
    raise NotImplementedError("write your pallas kernel here")


# speedup vs baseline: 1.4892x; 1.4892x over previous
import jax
import jax.numpy as jnp
from jax.experimental import pallas as pl
from jax.experimental.pallas import tpu as pltpu


def _stage1_kernel(x_ref, w1_ref, b1_ref, out_ref, *, ff_chunk: int):
    ff = w1_ref.shape[1]
    for j in range(ff // ff_chunk):
        sl = slice(j * ff_chunk, (j + 1) * ff_chunk)
        h = jnp.dot(x_ref[...], w1_ref[:, sl], preferred_element_type=jnp.float32)
        h = h + b1_ref[:, sl]
        g = jax.nn.gelu(h)
        out_ref[0, 0, sl] = jnp.sum(g, axis=0)


def _stage2_kernel(p_ref, s_ref, w2_ref, b2_ref, out_ref):
    inv_len = 1.0 / s_ref[:, 1:2].astype(jnp.float32)
    scaled = (p_ref[...] * inv_len).astype(jnp.bfloat16)
    out = jnp.dot(scaled, w2_ref[...], preferred_element_type=jnp.float32)
    out_ref[...] = out + b2_ref[...]


def kernel(slices, inputs, W1, b1, W2, b2):
    b = slices.shape[0]
    tot, d = inputs.shape
    seg = tot // b
    ff = W1.shape[1]

    x16 = inputs.astype(jnp.bfloat16)
    w1_16 = W1.astype(jnp.bfloat16)
    w2_16 = W2.astype(jnp.bfloat16)
    b1r = b1.reshape(1, ff)
    b2r = b2.reshape(1, d)

    import functools
    pooled = pl.pallas_call(
        functools.partial(_stage1_kernel, ff_chunk=2048),
        grid=(b,),
        in_specs=[
            pl.BlockSpec((seg, d), lambda i: (i, 0)),
            pl.BlockSpec((d, ff), lambda i: (0, 0)),
            pl.BlockSpec((1, ff), lambda i: (0, 0)),
        ],
        out_specs=pl.BlockSpec((1, 1, ff), lambda i: (i, 0, 0)),
        out_shape=jax.ShapeDtypeStruct((b, 1, ff), jnp.float32),
    )(x16, w1_16, b1r)
    pooled = pooled.reshape(b, ff)

    out = pl.pallas_call(
        _stage2_kernel,
        in_specs=[
            pl.BlockSpec((b, ff), lambda: (0, 0)),
            pl.BlockSpec((b, 2), lambda: (0, 0)),
            pl.BlockSpec((ff, d), lambda: (0, 0)),
            pl.BlockSpec((1, d), lambda: (0, 0)),
        ],
        out_specs=pl.BlockSpec((b, d), lambda: (0, 0)),
        out_shape=jax.ShapeDtypeStruct((b, d), jnp.float32),
    )(pooled, slices, w2_16, b2r)
    return out


# speedup vs baseline: 1.7457x; 1.1722x over previous
import functools

import jax
import jax.numpy as jnp
from jax.experimental import pallas as pl


def _stage1_kernel(x_ref, w1_ref, b1_ref, out_ref):
    x = x_ref[...].astype(jnp.bfloat16)
    w = w1_ref[...].astype(jnp.bfloat16)
    h = jnp.dot(x, w, preferred_element_type=jnp.float32)
    h = h + b1_ref[...]
    g = jax.nn.gelu(h)
    out_ref[0, 0, :] = jnp.sum(g, axis=0)


def _stage2_kernel(p_ref, s_ref, w2_ref, b2_ref, out_ref):
    k = pl.program_id(0)
    inv_len = 1.0 / s_ref[:, 1:2].astype(jnp.float32)
    scaled = ((p_ref[...] * inv_len)).astype(jnp.bfloat16)
    w = w2_ref[...].astype(jnp.bfloat16)
    part = jnp.dot(scaled, w, preferred_element_type=jnp.float32)

    @pl.when(k == 0)
    def _init():
        out_ref[...] = b2_ref[...] + part

    @pl.when(k != 0)
    def _acc():
        out_ref[...] += part


def kernel(slices, inputs, W1, b1, W2, b2):
    b = slices.shape[0]
    tot, d = inputs.shape
    seg = tot // b
    ff = W1.shape[1]

    ff_tile = 2048
    nj = ff // ff_tile
    b1r = b1.reshape(1, ff)
    b2r = b2.reshape(1, d)

    pooled = pl.pallas_call(
        _stage1_kernel,
        grid=(nj, b),
        in_specs=[
            pl.BlockSpec((seg, d), lambda j, i: (i, 0)),
            pl.BlockSpec((d, ff_tile), lambda j, i: (0, j)),
            pl.BlockSpec((1, ff_tile), lambda j, i: (0, j)),
        ],
        out_specs=pl.BlockSpec((1, 1, ff_tile), lambda j, i: (i, 0, j)),
        out_shape=jax.ShapeDtypeStruct((b, 1, ff), jnp.float32),
    )(inputs, W1, b1r)
    pooled = pooled.reshape(b, ff)

    k_tile = 2048
    nk = ff // k_tile
    out = pl.pallas_call(
        _stage2_kernel,
        grid=(nk,),
        in_specs=[
            pl.BlockSpec((b, k_tile), lambda k: (0, k)),
            pl.BlockSpec((b, 2), lambda k: (0, 0)),
            pl.BlockSpec((k_tile, d), lambda k: (k, 0)),
            pl.BlockSpec((1, d), lambda k: (0, 0)),
        ],
        out_specs=pl.BlockSpec((b, d), lambda k: (0, 0)),
        out_shape=jax.ShapeDtypeStruct((b, d), jnp.float32),
    )(pooled, slices, W2, b2r)
    return out


# speedup vs baseline: 1.9605x; 1.1231x over previous
import functools

import jax
import jax.numpy as jnp
from jax.experimental import pallas as pl


def _stage1_kernel(x_ref, w1_ref, b1_ref, out_ref):
    seg = x_ref.shape[0]
    x = x_ref[...].astype(jnp.bfloat16)
    w = w1_ref[...].astype(jnp.bfloat16)
    h = jnp.dot(x, w, preferred_element_type=jnp.float32)
    hb = (h + b1_ref[...]).astype(jnp.bfloat16)
    g = jax.nn.gelu(hb)
    ones = jnp.ones((1, seg), jnp.bfloat16)
    out_ref[0, 0, :] = jnp.dot(ones, g, preferred_element_type=jnp.float32)[0]


def _stage2_kernel(p_ref, s_ref, w2_ref, b2_ref, out_ref):
    k = pl.program_id(0)
    inv_len = 1.0 / s_ref[:, 1:2].astype(jnp.float32)
    scaled = ((p_ref[...] * inv_len)).astype(jnp.bfloat16)
    w = w2_ref[...].astype(jnp.bfloat16)
    part = jnp.dot(scaled, w, preferred_element_type=jnp.float32)

    @pl.when(k == 0)
    def _init():
        out_ref[...] = b2_ref[...] + part

    @pl.when(k != 0)
    def _acc():
        out_ref[...] += part


def kernel(slices, inputs, W1, b1, W2, b2):
    b = slices.shape[0]
    tot, d = inputs.shape
    seg = tot // b
    ff = W1.shape[1]

    ff_tile = 2048
    nj = ff // ff_tile
    b1r = b1.reshape(1, ff)
    b2r = b2.reshape(1, d)

    pooled = pl.pallas_call(
        _stage1_kernel,
        grid=(nj, b),
        in_specs=[
            pl.BlockSpec((seg, d), lambda j, i: (i, 0)),
            pl.BlockSpec((d, ff_tile), lambda j, i: (0, j)),
            pl.BlockSpec((1, ff_tile), lambda j, i: (0, j)),
        ],
        out_specs=pl.BlockSpec((1, 1, ff_tile), lambda j, i: (i, 0, j)),
        out_shape=jax.ShapeDtypeStruct((b, 1, ff), jnp.float32),
    )(inputs, W1, b1r)
    pooled = pooled.reshape(b, ff)

    k_tile = 2048
    nk = ff // k_tile
    out = pl.pallas_call(
        _stage2_kernel,
        grid=(nk,),
        in_specs=[
            pl.BlockSpec((b, k_tile), lambda k: (0, k)),
            pl.BlockSpec((b, 2), lambda k: (0, 0)),
            pl.BlockSpec((k_tile, d), lambda k: (k, 0)),
            pl.BlockSpec((1, d), lambda k: (0, 0)),
        ],
        out_specs=pl.BlockSpec((b, d), lambda k: (0, 0)),
        out_shape=jax.ShapeDtypeStruct((b, d), jnp.float32),
    )(pooled, slices, W2, b2r)
    return out
